# baseline (speedup 1.0000x reference)
import functools

import jax
import jax.numpy as jnp
from jax.experimental import pallas as pl
from jax.experimental.pallas import tpu as pltpu


def _mm_dma_kernel(ids_sref, ids_ref, head_ref, table_f32_ref,
                   table_bf16_ref, out_ref, gbuf, sem, *, chunks, dma_rows):
    i = pl.program_id(0)
    tb, D = head_ref.shape
    R = table_f32_ref.shape[0]
    mm = tb - dma_rows
    mc = mm // chunks
    base = i * tb + mm

    for r in range(dma_rows):
        idx = ids_sref[base + r]
        pltpu.make_async_copy(table_f32_ref.at[pl.ds(idx, 1)],
                              gbuf.at[pl.ds(r, 1)], sem).start()

    iota_mc = jax.lax.broadcasted_iota(jnp.int32, (mc, R), 1)
    table = table_bf16_ref[...]
    dots = []
    for k in range(chunks):
        ids_k = ids_ref[pl.ds(k * mc, mc), :]
        one_hot = (iota_mc == ids_k).astype(jnp.bfloat16)
        dots.append(jnp.dot(one_hot, table,
                            preferred_element_type=jnp.float32))
        if k >= 1:
            r0 = (k - 1) * mc
            out_ref[pl.ds(r0, mc), :] = head_ref[pl.ds(r0, mc), :] + dots[k - 1]
    r0 = (chunks - 1) * mc
    out_ref[pl.ds(r0, mc), :] = head_ref[pl.ds(r0, mc), :] + dots[-1]

    pltpu.make_async_copy(table_f32_ref.at[pl.ds(0, dma_rows)],
                          gbuf.at[pl.ds(0, dma_rows)], sem).wait()
    out_ref[pl.ds(mm, dma_rows), :] = head_ref[pl.ds(mm, dma_rows), :] + gbuf[...]


def kernel(head_embed, rel_ids, embed_table):
    B, D = head_embed.shape
    R, _ = embed_table.shape
    tb = 8192
    dma_rows = 1024
    chunks = 14
    grid_b = pl.cdiv(B, tb)

    ids_1d = rel_ids.astype(jnp.int32).reshape(B)
    ids_2d = ids_1d.reshape(B, 1)
    table_bf16 = embed_table.astype(jnp.bfloat16)
    body = functools.partial(_mm_dma_kernel, chunks=chunks, dma_rows=dma_rows)

    return pl.pallas_call(
        body,
        out_shape=jax.ShapeDtypeStruct((B, D), head_embed.dtype),
        grid_spec=pltpu.PrefetchScalarGridSpec(
            num_scalar_prefetch=1,
            grid=(grid_b,),
            in_specs=[
                pl.BlockSpec((tb, 1), lambda i, ids: (i, 0)),
                pl.BlockSpec((tb, D), lambda i, ids: (i, 0)),
                pl.BlockSpec((R, D), lambda i, ids: (0, 0)),
                pl.BlockSpec((R, D), lambda i, ids: (0, 0)),
            ],
            out_specs=pl.BlockSpec((tb, D), lambda i, ids: (i, 0)),
            scratch_shapes=[
                pltpu.VMEM((dma_rows, D), jnp.float32),
                pltpu.SemaphoreType.DMA(()),
            ],
        ),
        compiler_params=pltpu.CompilerParams(
            dimension_semantics=("parallel",),
        ),
    )(ids_1d, ids_2d, head_embed, embed_table, table_bf16)


# speedup vs baseline: 1.6266x; 1.6266x over previous
import functools

import jax
import jax.numpy as jnp
from jax.experimental import pallas as pl
from jax.experimental.pallas import tpu as pltpu

_TILE_CANDIDATES = (8192, 4096, 2048, 1024, 512, 256, 128, 64, 32, 16, 8)


def _onehot_matmul_kernel(ids_ref, head_ref, table_ref, out_ref, *, chunks):
    tb, D = head_ref.shape
    R = table_ref.shape[0]
    mc = tb // chunks
    iota_mc = jax.lax.broadcasted_iota(jnp.int32, (mc, R), 1)
    table = table_ref[...]
    dots = []
    for k in range(chunks):
        ids_k = ids_ref[pl.ds(k * mc, mc), :]
        one_hot = (iota_mc == ids_k).astype(jnp.bfloat16)
        dots.append(jnp.dot(one_hot, table,
                            preferred_element_type=jnp.float32))
        if k >= 1:
            r0 = (k - 1) * mc
            out_ref[pl.ds(r0, mc), :] = head_ref[pl.ds(r0, mc), :] + dots[k - 1]
    r0 = (chunks - 1) * mc
    out_ref[pl.ds(r0, mc), :] = head_ref[pl.ds(r0, mc), :] + dots[-1]


def kernel(head_embed, rel_ids, embed_table):
    B, D = head_embed.shape
    R, _ = embed_table.shape
    tb = next((t for t in _TILE_CANDIDATES if B % t == 0), min(B, 8))
    chunks = max(1, min(16, tb // 8))
    grid_b = pl.cdiv(B, tb)

    ids_2d = rel_ids.astype(jnp.int32).reshape(B, 1)
    table_bf16 = embed_table.astype(jnp.bfloat16)
    body = functools.partial(_onehot_matmul_kernel, chunks=chunks)

    return pl.pallas_call(
        body,
        out_shape=jax.ShapeDtypeStruct((B, D), head_embed.dtype),
        grid=(grid_b,),
        in_specs=[
            pl.BlockSpec((tb, 1), lambda i: (i, 0)),
            pl.BlockSpec((tb, D), lambda i: (i, 0)),
            pl.BlockSpec((R, D), lambda i: (0, 0)),
        ],
        out_specs=pl.BlockSpec((tb, D), lambda i: (i, 0)),
        compiler_params=pltpu.CompilerParams(
            dimension_semantics=("parallel",),
        ),
    )(ids_2d, head_embed, table_bf16)
